# Initial kernel scaffold; baseline (speedup 1.0000x reference)
import jax
import jax.numpy as jnp
from jax.experimental import pallas as pl


def kernel(y, context_ptr, graph_h, W, b):
    raise NotImplementedError("write your pallas kernel here")


# speedup vs baseline: 1.0422x; 1.0422x over previous
import functools

import jax
import jax.numpy as jnp
from jax import lax
from jax.experimental import pallas as pl
from jax.experimental.pallas import tpu as pltpu
from jax.experimental.pallas import tpu_sc as plsc

_N = 100000
_C = 256
_NIDX = 16
_CH = 112
_NCHUNK = -(-_N // _CH)
_NW = 32
_TRIPS = -(-_NCHUNK // _NW)


def _sc_body(y_hbm, idx_hbm, graph_hbm, w_hbm, b_hbm, out_hbm,
             buf, yb, w_v, b_v, idx_v):
    cid = lax.axis_index("c")
    sid = lax.axis_index("s")
    wid = sid * 2 + cid

    pltpu.sync_copy(w_hbm, w_v)
    pltpu.sync_copy(b_hbm, b_v)
    pltpu.sync_copy(idx_hbm, idx_v)

    idx16 = idx_v[...]
    iota16 = lax.iota(jnp.int32, 16)

    def chunk_body(t, carry):
        c = wid + t * _NW

        @pl.when(c < _NCHUNK)
        def _():
            pbase = jnp.minimum(c * _CH, _N - _CH)
            rbase = 2 * pbase
            pltpu.sync_copy(graph_hbm.at[pl.ds(rbase, 2 * _CH)], buf)
            pltpu.sync_copy(y_hbm.at[pl.ds(pbase, _CH)], yb)

            def group_body(g, gcarry):
                base = pbase + g * 16
                y16 = yb[pl.ds(g * 16, 16)]
                rowid = jnp.full((16,), base, jnp.int32) + iota16
                keep = rowid != jnp.full((16,), idx16[0], jnp.int32)
                for j in range(1, _NIDX):
                    keep = keep & (rowid != jnp.full((16,), idx16[j], jnp.int32))
                mf = jnp.where(keep, 1.0, 0.0).astype(jnp.float32)
                ym = y16 * mf
                for r in range(16):
                    ymv = jnp.full((16,), ym[r], jnp.float32)
                    mv = jnp.full((16,), mf[r], jnp.float32)
                    row = 2 * (g * 16 + r) + 1
                    for k in range(_C // 16):
                        sl = pl.ds(k * 16, 16)
                        buf[row, sl] = buf[row, sl] + ymv * w_v[sl] + mv * b_v[sl]
                return gcarry

            lax.fori_loop(0, _CH // 16, group_body, 0)
            pltpu.sync_copy(buf, out_hbm.at[pl.ds(rbase, 2 * _CH)])

        return carry

    lax.fori_loop(0, _TRIPS, chunk_body, 0)


_sc_call = functools.partial(
    pl.kernel,
    mesh=plsc.VectorSubcoreMesh(core_axis_name="c", subcore_axis_name="s"),
    out_type=jax.ShapeDtypeStruct((2 * _N, _C), jnp.float32),
    scratch_types=[
        pltpu.VMEM((2 * _CH, _C), jnp.float32),
        pltpu.VMEM((_CH,), jnp.float32),
        pltpu.VMEM((_C,), jnp.float32),
        pltpu.VMEM((_C,), jnp.float32),
        pltpu.VMEM((_NIDX,), jnp.int32),
    ],
)(_sc_body)


def kernel(y, context_ptr, graph_h, W, b):
    idx = context_ptr[1:] - 1
    eff = jnp.where(idx < 0, idx + _N, idx).astype(jnp.int32)
    w = W[:, 0]
    yf = y[:, 0]
    return _sc_call(yf, eff, graph_h, w, b)


# speedup vs baseline: 4.4919x; 4.3099x over previous
import functools

import jax
import jax.numpy as jnp
from jax import lax
from jax.experimental import pallas as pl
from jax.experimental.pallas import tpu as pltpu
from jax.experimental.pallas import tpu_sc as plsc

_N = 100000
_C = 256
_NIDX = 16
_CH = 80
_NCHUNK = -(-_N // _CH)
_NW = 32
_NBUF = 3
_TRIPS = -(-(-(-_NCHUNK // _NW)) // _NBUF) * _NBUF
_NGRP = _CH // 16


def _sc_body(y_hbm, idx_hbm, graph_hbm, w_hbm, b_hbm, out_hbm,
             buf0, buf1, buf2, yb0, yb1, yb2, w_v, b_v, idx_v,
             sg0, sg1, sg2, sy0, sy1, sy2, so0, so1, so2):
    cid = lax.axis_index("c")
    sid = lax.axis_index("s")
    wid = sid * 2 + cid

    bufs = (buf0, buf1, buf2)
    ybs = (yb0, yb1, yb2)
    sgs = (sg0, sg1, sg2)
    sys_ = (sy0, sy1, sy2)
    sos = (so0, so1, so2)

    pltpu.sync_copy(w_hbm, w_v)
    pltpu.sync_copy(b_hbm, b_v)
    pltpu.sync_copy(idx_hbm, idx_v)

    idx16 = idx_v[...]
    iota16 = lax.iota(jnp.int32, 16)
    wk_regs = [w_v[pl.ds(k * 16, 16)] for k in range(_C // 16)]
    bk_regs = [b_v[pl.ds(k * 16, 16)] for k in range(_C // 16)]

    def pbase_of(c):
        return jnp.minimum(c * _CH, _N - _CH)

    def start_in(p, c):
        pb = pbase_of(c)
        pltpu.make_async_copy(
            graph_hbm.at[pl.ds(2 * pb, 2 * _CH)], bufs[p], sgs[p]).start()
        pltpu.make_async_copy(
            y_hbm.at[pl.ds(pb, _CH)], ybs[p], sys_[p]).start()

    def wait_in(p):
        pltpu.make_async_copy(
            graph_hbm.at[pl.ds(0, 2 * _CH)], bufs[p], sgs[p]).wait()
        pltpu.make_async_copy(
            y_hbm.at[pl.ds(0, _CH)], ybs[p], sys_[p]).wait()

    def start_out(p, c):
        pb = pbase_of(c)
        pltpu.make_async_copy(
            bufs[p], out_hbm.at[pl.ds(2 * pb, 2 * _CH)], sos[p]).start()

    def wait_out(p):
        pltpu.make_async_copy(
            bufs[p], out_hbm.at[pl.ds(0, 2 * _CH)], sos[p]).wait()

    def compute(p, c):
        buf = bufs[p]
        yb = ybs[p]
        pbase = pbase_of(c)

        def group_body(g, gcarry):
            y16 = yb[pl.ds(g * 16, 16)]
            rowid = jnp.full((16,), pbase + g * 16, jnp.int32) + iota16
            keep = rowid != jnp.full((16,), idx16[0], jnp.int32)
            for j in range(1, _NIDX):
                keep = keep & (rowid != jnp.full((16,), idx16[j], jnp.int32))
            mf = jnp.where(keep, 1.0, 0.0).astype(jnp.float32)
            ym = y16 * mf
            for r in range(16):
                ymv = jnp.full((16,), ym[r], jnp.float32)
                mv = jnp.full((16,), mf[r], jnp.float32)
                row = 2 * (g * 16 + r) + 1
                vals = [buf[row, pl.ds(k * 16, 16)] for k in range(_C // 16)]
                for k in range(_C // 16):
                    buf[row, pl.ds(k * 16, 16)] = (
                        vals[k] + ymv * wk_regs[k] + mv * bk_regs[k])
            return gcarry

        lax.fori_loop(0, _NGRP, group_body, 0)

    start_in(0, wid)

    def outer_body(u, carry):
        for p in range(_NBUF):
            t = u * _NBUF + p
            c_cur = wid + t * _NW
            c_nxt = c_cur + _NW
            pn = (p + 1) % _NBUF

            @pl.when(c_nxt < _NCHUNK)
            def _():
                @pl.when(t + 1 >= _NBUF)
                def _():
                    wait_out(pn)

                start_in(pn, c_nxt)

            @pl.when(c_cur < _NCHUNK)
            def _():
                wait_in(p)
                compute(p, c_cur)
                start_out(p, c_cur)

        return carry

    lax.fori_loop(0, _TRIPS // _NBUF, outer_body, 0)

    for p in range(_NBUF):
        wait_out(p)


_sc_call = functools.partial(
    pl.kernel,
    mesh=plsc.VectorSubcoreMesh(core_axis_name="c", subcore_axis_name="s"),
    out_type=jax.ShapeDtypeStruct((2 * _N, _C), jnp.float32),
    scratch_types=[
        pltpu.VMEM((2 * _CH, _C), jnp.float32),
        pltpu.VMEM((2 * _CH, _C), jnp.float32),
        pltpu.VMEM((2 * _CH, _C), jnp.float32),
        pltpu.VMEM((_CH,), jnp.float32),
        pltpu.VMEM((_CH,), jnp.float32),
        pltpu.VMEM((_CH,), jnp.float32),
        pltpu.VMEM((_C,), jnp.float32),
        pltpu.VMEM((_C,), jnp.float32),
        pltpu.VMEM((_NIDX,), jnp.int32),
        pltpu.SemaphoreType.DMA,
        pltpu.SemaphoreType.DMA,
        pltpu.SemaphoreType.DMA,
        pltpu.SemaphoreType.DMA,
        pltpu.SemaphoreType.DMA,
        pltpu.SemaphoreType.DMA,
        pltpu.SemaphoreType.DMA,
        pltpu.SemaphoreType.DMA,
        pltpu.SemaphoreType.DMA,
    ],
)(_sc_body)


def kernel(y, context_ptr, graph_h, W, b):
    idx = context_ptr[1:] - 1
    eff = jnp.where(idx < 0, idx + _N, idx).astype(jnp.int32)
    w = W[:, 0]
    yf = y[:, 0]
    return _sc_call(yf, eff, graph_h, w, b)


# speedup vs baseline: 4.5787x; 1.0193x over previous
import functools

import jax
import jax.numpy as jnp
from jax import lax
from jax.experimental import pallas as pl
from jax.experimental.pallas import tpu as pltpu
from jax.experimental.pallas import tpu_sc as plsc

_N = 100000
_C = 256
_NIDX = 16
_CH = 80
_NCHUNK = -(-_N // _CH)
_NW = 32
_NBUF = 3
_TRIPS = -(-(-(-_NCHUNK // _NW)) // _NBUF) * _NBUF
_NGRP = _CH // 16


def _sc_body(y_hbm, idx_hbm, graph_hbm, w_hbm, b_hbm, out_hbm,
             buf0, buf1, buf2, yb0, yb1, yb2, w_v, b_v, idx_v,
             sg0, sg1, sg2, sy0, sy1, sy2, so0, so1, so2):
    cid = lax.axis_index("c")
    sid = lax.axis_index("s")
    wid = sid * 2 + cid

    bufs = (buf0, buf1, buf2)
    ybs = (yb0, yb1, yb2)
    sgs = (sg0, sg1, sg2)
    sys_ = (sy0, sy1, sy2)
    sos = (so0, so1, so2)

    pltpu.sync_copy(w_hbm, w_v)
    pltpu.sync_copy(b_hbm, b_v)
    pltpu.sync_copy(idx_hbm, idx_v)

    idx16 = idx_v[...]
    iota16 = lax.iota(jnp.int32, 16)

    def pbase_of(c):
        return jnp.minimum(c * _CH, _N - _CH)

    def start_in(p, c):
        pb = pbase_of(c)
        pltpu.make_async_copy(
            graph_hbm.at[pl.ds(2 * pb, 2 * _CH)], bufs[p], sgs[p]).start()
        pltpu.make_async_copy(
            y_hbm.at[pl.ds(pb, _CH)], ybs[p], sys_[p]).start()

    def wait_in(p):
        pltpu.make_async_copy(
            graph_hbm.at[pl.ds(0, 2 * _CH)], bufs[p], sgs[p]).wait()
        pltpu.make_async_copy(
            y_hbm.at[pl.ds(0, _CH)], ybs[p], sys_[p]).wait()

    def start_out(p, c):
        pb = pbase_of(c)
        pltpu.make_async_copy(
            bufs[p], out_hbm.at[pl.ds(2 * pb, 2 * _CH)], sos[p]).start()

    def wait_out(p):
        pltpu.make_async_copy(
            bufs[p], out_hbm.at[pl.ds(0, 2 * _CH)], sos[p]).wait()

    def compute(p, c):
        buf = bufs[p]
        yb = ybs[p]
        pbase = pbase_of(c)

        def group_body(g, gcarry):
            y16 = yb[pl.ds(g * 16, 16)]
            rowid = jnp.full((16,), pbase + g * 16, jnp.int32) + iota16
            keep = rowid != jnp.full((16,), idx16[0], jnp.int32)
            for j in range(1, _NIDX):
                keep = keep & (rowid != jnp.full((16,), idx16[j], jnp.int32))
            mf = jnp.where(keep, 1.0, 0.0).astype(jnp.float32)
            ym = y16 * mf
            for h in range(2):
                ks = range(h * 8, h * 8 + 8)
                wk_h = [w_v[pl.ds(k * 16, 16)] for k in ks]
                bk_h = [b_v[pl.ds(k * 16, 16)] for k in ks]
                for r in range(16):
                    ymv = jnp.full((16,), ym[r], jnp.float32)
                    mv = jnp.full((16,), mf[r], jnp.float32)
                    row = 2 * (g * 16 + r) + 1
                    vals = [buf[row, pl.ds(k * 16, 16)] for k in ks]
                    for i, k in enumerate(ks):
                        buf[row, pl.ds(k * 16, 16)] = (
                            vals[i] + ymv * wk_h[i] + mv * bk_h[i])
            return gcarry

        lax.fori_loop(0, _NGRP, group_body, 0)

    start_in(0, wid)

    def outer_body(u, carry):
        for p in range(_NBUF):
            t = u * _NBUF + p
            c_cur = wid + t * _NW
            c_nxt = c_cur + _NW
            pn = (p + 1) % _NBUF

            @pl.when(c_nxt < _NCHUNK)
            def _():
                @pl.when(t + 1 >= _NBUF)
                def _():
                    wait_out(pn)

                start_in(pn, c_nxt)

            @pl.when(c_cur < _NCHUNK)
            def _():
                wait_in(p)
                compute(p, c_cur)
                start_out(p, c_cur)

        return carry

    lax.fori_loop(0, _TRIPS // _NBUF, outer_body, 0)

    for p in range(_NBUF):
        wait_out(p)


_sc_call = functools.partial(
    pl.kernel,
    mesh=plsc.VectorSubcoreMesh(core_axis_name="c", subcore_axis_name="s"),
    out_type=jax.ShapeDtypeStruct((2 * _N, _C), jnp.float32),
    scratch_types=[
        pltpu.VMEM((2 * _CH, _C), jnp.float32),
        pltpu.VMEM((2 * _CH, _C), jnp.float32),
        pltpu.VMEM((2 * _CH, _C), jnp.float32),
        pltpu.VMEM((_CH,), jnp.float32),
        pltpu.VMEM((_CH,), jnp.float32),
        pltpu.VMEM((_CH,), jnp.float32),
        pltpu.VMEM((_C,), jnp.float32),
        pltpu.VMEM((_C,), jnp.float32),
        pltpu.VMEM((_NIDX,), jnp.int32),
        pltpu.SemaphoreType.DMA,
        pltpu.SemaphoreType.DMA,
        pltpu.SemaphoreType.DMA,
        pltpu.SemaphoreType.DMA,
        pltpu.SemaphoreType.DMA,
        pltpu.SemaphoreType.DMA,
        pltpu.SemaphoreType.DMA,
        pltpu.SemaphoreType.DMA,
        pltpu.SemaphoreType.DMA,
    ],
)(_sc_body)


def kernel(y, context_ptr, graph_h, W, b):
    idx = context_ptr[1:] - 1
    eff = jnp.where(idx < 0, idx + _N, idx).astype(jnp.int32)
    w = W[:, 0]
    yf = y[:, 0]
    return _sc_call(yf, eff, graph_h, w, b)


# speedup vs baseline: 4.5879x; 1.0020x over previous
import functools

import jax
import jax.numpy as jnp
from jax import lax
from jax.experimental import pallas as pl
from jax.experimental.pallas import tpu as pltpu
from jax.experimental.pallas import tpu_sc as plsc

_N = 100000
_C = 256
_NIDX = 16
_CH = 48
_NCHUNK = -(-_N // _CH)
_NW = 32
_NBUF = 4
_PREF = 2
_TRIPS = -(-(-(-_NCHUNK // _NW)) // _NBUF) * _NBUF
_NGRP = _CH // 16


def _sc_body(y_hbm, idx_hbm, graph_hbm, w_hbm, b_hbm, out_hbm,
             buf0, buf1, buf2, buf3, yb0, yb1, yb2, yb3, w_v, b_v, idx_v,
             sg0, sg1, sg2, sg3, sy0, sy1, sy2, sy3, so0, so1, so2, so3):
    cid = lax.axis_index("c")
    sid = lax.axis_index("s")
    wid = sid * 2 + cid

    bufs = (buf0, buf1, buf2, buf3)
    ybs = (yb0, yb1, yb2, yb3)
    sgs = (sg0, sg1, sg2, sg3)
    sys_ = (sy0, sy1, sy2, sy3)
    sos = (so0, so1, so2, so3)

    pltpu.sync_copy(w_hbm, w_v)
    pltpu.sync_copy(b_hbm, b_v)
    pltpu.sync_copy(idx_hbm, idx_v)

    idx16 = idx_v[...]
    iota16 = lax.iota(jnp.int32, 16)

    def pbase_of(c):
        return jnp.minimum(c * _CH, _N - _CH)

    def start_in(p, c):
        pb = pbase_of(c)
        pltpu.make_async_copy(
            graph_hbm.at[pl.ds(2 * pb, 2 * _CH)], bufs[p], sgs[p]).start()
        pltpu.make_async_copy(
            y_hbm.at[pl.ds(pb, _CH)], ybs[p], sys_[p]).start()

    def wait_in(p):
        pltpu.make_async_copy(
            graph_hbm.at[pl.ds(0, 2 * _CH)], bufs[p], sgs[p]).wait()
        pltpu.make_async_copy(
            y_hbm.at[pl.ds(0, _CH)], ybs[p], sys_[p]).wait()

    def start_out(p, c):
        pb = pbase_of(c)
        pltpu.make_async_copy(
            bufs[p], out_hbm.at[pl.ds(2 * pb, 2 * _CH)], sos[p]).start()

    def wait_out(p):
        pltpu.make_async_copy(
            bufs[p], out_hbm.at[pl.ds(0, 2 * _CH)], sos[p]).wait()

    def compute(p, c):
        buf = bufs[p]
        yb = ybs[p]
        pbase = pbase_of(c)

        def group_body(g, gcarry):
            y16 = yb[pl.ds(g * 16, 16)]
            rowid = jnp.full((16,), pbase + g * 16, jnp.int32) + iota16
            keep = rowid != jnp.full((16,), idx16[0], jnp.int32)
            for j in range(1, _NIDX):
                keep = keep & (rowid != jnp.full((16,), idx16[j], jnp.int32))
            mf = jnp.where(keep, 1.0, 0.0).astype(jnp.float32)
            ym = y16 * mf
            for h in range(2):
                ks = range(h * 8, h * 8 + 8)
                wk_h = [w_v[pl.ds(k * 16, 16)] for k in ks]
                bk_h = [b_v[pl.ds(k * 16, 16)] for k in ks]
                for r in range(16):
                    ymv = jnp.full((16,), ym[r], jnp.float32)
                    mv = jnp.full((16,), mf[r], jnp.float32)
                    row = 2 * (g * 16 + r) + 1
                    vals = [buf[row, pl.ds(k * 16, 16)] for k in ks]
                    for i, k in enumerate(ks):
                        buf[row, pl.ds(k * 16, 16)] = (
                            vals[i] + ymv * wk_h[i] + mv * bk_h[i])
            return gcarry

        lax.fori_loop(0, _NGRP, group_body, 0)

    for s in range(_PREF):
        start_in(s, wid + s * _NW)

    def outer_body(u, carry):
        for p in range(_NBUF):
            t = u * _NBUF + p
            c_cur = wid + t * _NW
            c_pre = c_cur + _PREF * _NW
            pn = (p + _PREF) % _NBUF

            @pl.when(c_pre < _NCHUNK)
            def _():
                @pl.when(t + _PREF >= _NBUF)
                def _():
                    wait_out(pn)

                start_in(pn, c_pre)

            @pl.when(c_cur < _NCHUNK)
            def _():
                wait_in(p)
                compute(p, c_cur)
                start_out(p, c_cur)

        return carry

    lax.fori_loop(0, _TRIPS // _NBUF, outer_body, 0)

    for p in range(_NBUF):
        wait_out(p)


_sc_call = functools.partial(
    pl.kernel,
    mesh=plsc.VectorSubcoreMesh(core_axis_name="c", subcore_axis_name="s"),
    out_type=jax.ShapeDtypeStruct((2 * _N, _C), jnp.float32),
    scratch_types=(
        [pltpu.VMEM((2 * _CH, _C), jnp.float32)] * _NBUF
        + [pltpu.VMEM((_CH,), jnp.float32)] * _NBUF
        + [pltpu.VMEM((_C,), jnp.float32)] * 2
        + [pltpu.VMEM((_NIDX,), jnp.int32)]
        + [pltpu.SemaphoreType.DMA] * (3 * _NBUF)
    ),
)(_sc_body)


def kernel(y, context_ptr, graph_h, W, b):
    idx = context_ptr[1:] - 1
    eff = jnp.where(idx < 0, idx + _N, idx).astype(jnp.int32)
    w = W[:, 0]
    yf = y[:, 0]
    return _sc_call(yf, eff, graph_h, w, b)
